# Initial kernel scaffold; baseline (speedup 1.0000x reference)
import jax
import jax.numpy as jnp
from jax.experimental import pallas as pl


def kernel(x, edge_index, edge_weights, W1, b1, W2, b2, W3, b3):
    raise NotImplementedError("write your pallas kernel here")


# speedup vs baseline: 8.4869x; 8.4869x over previous
import functools

import jax
import jax.numpy as jnp
from jax import lax
from jax.experimental import pallas as pl
from jax.experimental.pallas import tpu as pltpu
from jax.experimental.pallas import tpu_sc as plsc

N = 10000
D = 128
NC = 2
NS = 16
NW = NC * NS
C = 128
LANES = 16
NPAD = 10240
ZROWS = NPAD // NS

_MESH = plsc.VectorSubcoreMesh(
    core_axis_name="c", subcore_axis_name="s", num_cores=NC, num_subcores=NS)
_SC_PARAMS = pltpu.CompilerParams(needs_layout_passes=False)


def _num_chunks(e_total):
    per_tile = -(-e_total // NW)
    return -(-per_tile // C)


def _deg_call(dstp, ewp, nch):
    @functools.partial(
        pl.kernel,
        out_type=jax.ShapeDtypeStruct((NW, N), jnp.float32),
        mesh=_MESH,
        compiler_params=_SC_PARAMS,
        scratch_types=[
            pltpu.VMEM((nch, C), jnp.int32),
            pltpu.VMEM((nch, C), jnp.float32),
            pltpu.VMEM((N,), jnp.float32),
        ],
    )
    def deg_kernel(dst_hbm, ew_hbm, out_hbm, dst_v, ew_v, deg_v):
        c = lax.axis_index("c")
        s = lax.axis_index("s")
        wid = c * NS + s
        pltpu.sync_copy(dst_hbm.at[wid], dst_v)
        pltpu.sync_copy(ew_hbm.at[wid], ew_v)

        @pl.loop(0, N // LANES)
        def _zero(i):
            deg_v[pl.ds(i * LANES, LANES)] = jnp.zeros((LANES,), jnp.float32)

        lane = lax.iota(jnp.int32, LANES)

        @pl.loop(0, nch)
        def _chunk(j):
            for g in range(C // LANES):
                dstv = dst_v[j, pl.ds(g * LANES, LANES)]
                ewv = ew_v[j, pl.ds(g * LANES, LANES)]
                for l in range(LANES):
                    plsc.addupdate_scatter(deg_v, [dstv], ewv,
                                           mask=lane == l)

        pltpu.sync_copy(deg_v, out_hbm.at[wid])

    return deg_kernel(dstp, ewp)


def _edge_call(y, srcp, dstp, ewp, nch):
    @functools.partial(
        pl.kernel,
        out_type=jax.ShapeDtypeStruct((NC, NPAD, D), jnp.float32),
        mesh=_MESH,
        compiler_params=_SC_PARAMS,
        scratch_types=[
            pltpu.VMEM((nch, C), jnp.int32),
            pltpu.VMEM((nch, C), jnp.int32),
            pltpu.VMEM((nch, C), jnp.float32),
            pltpu.VMEM((C, D), jnp.float32),
            pltpu.VMEM_SHARED((NPAD, D), jnp.float32),
        ],
    )
    def edge_kernel(y_hbm, src_hbm, dst_hbm, ew_hbm, out_hbm,
                    src_v, dst_v, ew_v, buf, acc):
        c = lax.axis_index("c")
        s = lax.axis_index("s")
        wid = c * NS + s
        pltpu.sync_copy(src_hbm.at[wid], src_v)
        pltpu.sync_copy(dst_hbm.at[wid], dst_v)
        pltpu.sync_copy(ew_hbm.at[wid], ew_v)

        @pl.loop(0, C)
        def _zbuf(i):
            for g in range(D // LANES):
                buf[i, pl.ds(g * LANES, LANES)] = jnp.zeros((LANES,),
                                                            jnp.float32)

        base = s * ZROWS
        for r in range(0, ZROWS, C):
            rows = min(C, ZROWS - r)
            pltpu.sync_copy(buf.at[pl.ds(0, rows)],
                            acc.at[pl.ds(base + r, rows)])
        plsc.subcore_barrier()

        @pl.loop(0, nch)
        def _chunk(j):
            pltpu.sync_copy(y_hbm.at[src_v.at[j]], buf)

            @pl.loop(0, C)
            def _edge(e):
                i16 = jnp.broadcast_to(j, (LANES,)).astype(jnp.int32)
                e16 = jnp.broadcast_to(e, (LANES,)).astype(jnp.int32)
                w = plsc.load_gather(ew_v, [i16, e16])
                for g in range(D // LANES):
                    sl = pl.ds(g * LANES, LANES)
                    buf[e, sl] = buf[e, sl] * w

            pltpu.sync_copy(buf, acc.at[dst_v.at[j]], add=True)

        plsc.subcore_barrier()
        pltpu.sync_copy(acc.at[pl.ds(base, ZROWS)],
                        out_hbm.at[c].at[pl.ds(base, ZROWS)])

    return edge_kernel(y, srcp, dstp, ewp)


_ROWS_BLK = 1000
_GRID = N // _ROWS_BLK


def _tc_dinv(degp):
    def body(degp_ref, dinv_ref):
        deg = jnp.sum(degp_ref[...], axis=0) + 1.0
        dinv_ref[...] = lax.rsqrt(deg)[:, None]

    return pl.pallas_call(
        body,
        out_shape=jax.ShapeDtypeStruct((N, 1), jnp.float32),
    )(degp)


def _tc_first(x, w, dinv):
    def body(x_ref, w_ref, dinv_ref, y_ref):
        xw = jnp.dot(x_ref[...], w_ref[...],
                     preferred_element_type=jnp.float32)
        y_ref[...] = dinv_ref[...] * xw

    return pl.pallas_call(
        body,
        grid=(_GRID,),
        in_specs=[
            pl.BlockSpec((_ROWS_BLK, D), lambda i: (i, 0)),
            pl.BlockSpec((D, D), lambda i: (0, 0)),
            pl.BlockSpec((_ROWS_BLK, 1), lambda i: (i, 0)),
        ],
        out_specs=pl.BlockSpec((_ROWS_BLK, D), lambda i: (i, 0)),
        out_shape=jax.ShapeDtypeStruct((N, D), jnp.float32),
    )(x, w, dinv)


def _tc_mid(parts, y_prev, dinv, b, w_next):
    def body(p_ref, y_ref, dinv_ref, b_ref, w_ref, o_ref):
        dinv = dinv_ref[...]
        pre = dinv * (p_ref[0] + p_ref[1] + y_ref[...]) + b_ref[...]
        h = jnp.maximum(pre, 0.0)
        o_ref[...] = dinv * jnp.dot(h, w_ref[...],
                                    preferred_element_type=jnp.float32)

    return pl.pallas_call(
        body,
        grid=(_GRID,),
        in_specs=[
            pl.BlockSpec((NC, _ROWS_BLK, D), lambda i: (0, i, 0)),
            pl.BlockSpec((_ROWS_BLK, D), lambda i: (i, 0)),
            pl.BlockSpec((_ROWS_BLK, 1), lambda i: (i, 0)),
            pl.BlockSpec((1, D), lambda i: (0, 0)),
            pl.BlockSpec((D, D), lambda i: (0, 0)),
        ],
        out_specs=pl.BlockSpec((_ROWS_BLK, D), lambda i: (i, 0)),
        out_shape=jax.ShapeDtypeStruct((N, D), jnp.float32),
    )(parts, y_prev, dinv, b, w_next)


def _tc_last(parts, y_prev, dinv, b):
    def body(p_ref, y_ref, dinv_ref, b_ref, o_ref):
        dinv = dinv_ref[...]
        o_ref[...] = dinv * (p_ref[0] + p_ref[1] + y_ref[...]) + b_ref[...]

    return pl.pallas_call(
        body,
        grid=(_GRID,),
        in_specs=[
            pl.BlockSpec((NC, _ROWS_BLK, D), lambda i: (0, i, 0)),
            pl.BlockSpec((_ROWS_BLK, D), lambda i: (i, 0)),
            pl.BlockSpec((_ROWS_BLK, 1), lambda i: (i, 0)),
            pl.BlockSpec((1, D), lambda i: (0, 0)),
        ],
        out_specs=pl.BlockSpec((_ROWS_BLK, D), lambda i: (i, 0)),
        out_shape=jax.ShapeDtypeStruct((N, D), jnp.float32),
    )(parts, y_prev, dinv, b)


def kernel(x, edge_index, edge_weights, W1, b1, W2, b2, W3, b3):
    e_total = edge_index.shape[1]
    nch = _num_chunks(e_total)
    e_pad = NW * nch * C
    pad = e_pad - e_total

    src = jnp.pad(edge_index[0], (0, pad)).reshape(NW, nch, C)
    dst = jnp.pad(edge_index[1], (0, pad)).reshape(NW, nch, C)
    ew = jnp.pad(edge_weights, (0, pad)).reshape(NW, nch, C)

    degp = _deg_call(dst, ew, nch)
    dinv = _tc_dinv(degp)
    y1 = _tc_first(x, W1, dinv)
    b1r = b1.reshape(1, D)
    b2r = b2.reshape(1, D)
    b3r = b3.reshape(1, D)

    parts1 = _edge_call(y1, src, dst, ew, nch)
    y2 = _tc_mid(parts1, y1, dinv, b1r, W2)
    parts2 = _edge_call(y2, src, dst, ew, nch)
    y3 = _tc_mid(parts2, y2, dinv, b2r, W3)
    parts3 = _edge_call(y3, src, dst, ew, nch)
    return _tc_last(parts3, y3, dinv, b3r)
